# baseline (speedup 1.0000x reference)
import functools

import jax
import jax.numpy as jnp
from jax import lax
from jax.experimental import pallas as pl
from jax.experimental.pallas import tpu as pltpu
from jax.experimental.pallas import tpu_sc as plsc

VOCAB = 100000
D = 768
B = 4
S = 2048

NW = 32
P = S // NW
CHUNK = 32
NCHUNK = (B * P) // CHUNK
LANES = 16
COLV = D // LANES

_mesh = plsc.VectorSubcoreMesh(core_axis_name="c", subcore_axis_name="s")


NBUF = 3


@functools.partial(
    pl.kernel,
    mesh=_mesh,
    out_type=jax.ShapeDtypeStruct((B * S, D), jnp.float32),
    scratch_types=[
        pltpu.VMEM((B * P,), jnp.int32),
        pltpu.VMEM((P, D), jnp.float32),
        pltpu.VMEM((CHUNK, D), jnp.float32),
        pltpu.VMEM((CHUNK, D), jnp.float32),
        pltpu.VMEM((CHUNK, D), jnp.float32),
        pltpu.SemaphoreType.DMA,
        pltpu.SemaphoreType.DMA,
        pltpu.SemaphoreType.DMA,
        pltpu.SemaphoreType.DMA,
        pltpu.SemaphoreType.DMA,
        pltpu.SemaphoreType.DMA,
        pltpu.SemaphoreType.DMA,
        pltpu.SemaphoreType.DMA,
        pltpu.SemaphoreType.DMA,
        pltpu.SemaphoreType.DMA,
        pltpu.SemaphoreType.DMA,
        pltpu.SemaphoreType.DMA,
    ],
)
def _emb_kernel(x_hbm, tok_hbm, pos_hbm, out_hbm,
                idx_v, pos_v, tok0, tok1, tok2,
                gsem0, gsem1, gsem2, wsem0, wsem1, wsem2, psem0, psem1,
                isem0, isem1, isem2, isem3):
    cid = lax.axis_index("c")
    sid = lax.axis_index("s")
    wid = sid * 2 + cid
    p0 = wid * P

    isems = (isem0, isem1, isem2, isem3)
    idx_hs = [pltpu.async_copy(x_hbm.at[b, pl.ds(p0, P)],
                               idx_v.at[pl.ds(b * P, P)], isems[b])
              for b in range(B)]
    HALF = P // 2
    pos_h0 = pltpu.async_copy(pos_hbm.at[pl.ds(p0, HALF)],
                              pos_v.at[pl.ds(0, HALF), :], psem0)
    pos_h1 = pltpu.async_copy(pos_hbm.at[pl.ds(p0 + HALF, HALF)],
                              pos_v.at[pl.ds(HALF, HALF), :], psem1)
    for h_ in idx_hs:
        h_.wait()

    bufs = (tok0, tok1, tok2)
    gsems = (gsem0, gsem1, gsem2)
    wsems = (wsem0, wsem1, wsem2)
    wb = [None] * NBUF
    gh = [None] * NCHUNK

    CHUNKS = [(b, h) for h in range(P // CHUNK) for b in range(B)]

    def start_gather(j):
        k = j % NBUF
        if wb[k] is not None:
            wb[k].wait()
        b, h = CHUNKS[j]
        return pltpu.async_copy(
            tok_hbm.at[idx_v.at[pl.ds((b * (P // CHUNK) + h) * CHUNK, CHUNK)]],
            bufs[k], gsems[k])

    for j in range(NBUF):
        gh[j] = start_gather(j)
    pos_h0.wait()

    for g in range(NCHUNK):
        if g == NCHUNK // 2:
            pos_h1.wait()
        gh[g].wait()
        k = g % NBUF
        buf = bufs[k]
        b, h = CHUNKS[g]
        prow0 = h * CHUNK

        @plsc.parallel_loop(0, CHUNK * COLV, 1, unroll=8)
        def _(i, buf=buf, prow0=prow0):
            sl = pl.ds(pl.multiple_of((i >> 5) << 4, LANES), LANES)
            r = i & (CHUNK - 1)
            plsc.addupdate(buf.at[r, sl], pos_v[prow0 + r, sl])

        out_base = b * S + p0 + h * CHUNK
        wb[k] = pltpu.async_copy(buf, out_hbm.at[pl.ds(out_base, CHUNK)],
                                 wsems[k])
        if g + NBUF < NCHUNK:
            gh[g + NBUF] = start_gather(g + NBUF)

    for k in range(NBUF):
        wb[k].wait()


def kernel(x, tok_table, pos_table):
    out = _emb_kernel(x.astype(jnp.int32), tok_table, pos_table)
    return out.reshape(B, S, D)


# speedup vs baseline: 1.0634x; 1.0634x over previous
import functools

import jax
import jax.numpy as jnp
from jax import lax
from jax.experimental import pallas as pl
from jax.experimental.pallas import tpu as pltpu
from jax.experimental.pallas import tpu_sc as plsc

VOCAB = 100000
D = 768
B = 4
S = 2048

NW = 32
P = S // NW
CHUNK = 32
NCHUNK = (B * P) // CHUNK
LANES = 16
COLV = D // LANES

_mesh = plsc.VectorSubcoreMesh(core_axis_name="c", subcore_axis_name="s")


NBUF = 3


@functools.partial(
    pl.kernel,
    mesh=_mesh,
    out_type=jax.ShapeDtypeStruct((B * S, D), jnp.float32),
    scratch_types=[
        pltpu.VMEM((B * P,), jnp.int32),
        pltpu.VMEM((P, D), jnp.float32),
        pltpu.VMEM((CHUNK, D), jnp.float32),
        pltpu.VMEM((CHUNK, D), jnp.float32),
        pltpu.VMEM((CHUNK, D), jnp.float32),
        pltpu.SemaphoreType.DMA,
        pltpu.SemaphoreType.DMA,
        pltpu.SemaphoreType.DMA,
        pltpu.SemaphoreType.DMA,
        pltpu.SemaphoreType.DMA,
        pltpu.SemaphoreType.DMA,
        pltpu.SemaphoreType.DMA,
        pltpu.SemaphoreType.DMA,
        pltpu.SemaphoreType.DMA,
        pltpu.SemaphoreType.DMA,
        pltpu.SemaphoreType.DMA,
        pltpu.SemaphoreType.DMA,
    ],
)
def _emb_kernel(x_hbm, tok_hbm, pos_hbm, out_hbm,
                idx_v, pos_v, tok0, tok1, tok2,
                gsem0, gsem1, gsem2, wsem0, wsem1, wsem2, psem0, psem1,
                isem0, isem1, isem2, isem3):
    cid = lax.axis_index("c")
    sid = lax.axis_index("s")
    wid = sid * 2 + cid
    p0 = wid * P

    isems = (isem0, isem1, isem2, isem3)
    idx_hs = [pltpu.async_copy(x_hbm.at[b, pl.ds(p0, P)],
                               idx_v.at[pl.ds(b * P, P)], isems[b])
              for b in range(B)]
    HALF = P // 2
    pos_h0 = pltpu.async_copy(pos_hbm.at[pl.ds(p0, HALF)],
                              pos_v.at[pl.ds(0, HALF), :], psem0)
    pos_h1 = pltpu.async_copy(pos_hbm.at[pl.ds(p0 + HALF, HALF)],
                              pos_v.at[pl.ds(HALF, HALF), :], psem1)
    for h_ in idx_hs:
        h_.wait()

    bufs = (tok0, tok1, tok2)
    gsems = (gsem0, gsem1, gsem2)
    wsems = (wsem0, wsem1, wsem2)
    wb = [None] * NBUF
    gh = [None] * NCHUNK

    CHUNKS = [(b, h) for h in range(P // CHUNK) for b in range(B)]

    def start_gather(j):
        k = j % NBUF
        if wb[k] is not None:
            wb[k].wait()
        b, h = CHUNKS[j]
        return pltpu.async_copy(
            tok_hbm.at[idx_v.at[pl.ds((b * (P // CHUNK) + h) * CHUNK, CHUNK)]],
            bufs[k], gsems[k])

    for j in range(NBUF - 1):
        gh[j] = start_gather(j)
    pos_h0.wait()

    for g in range(NCHUNK):
        if g == NCHUNK // 2:
            pos_h1.wait()
        gh[g].wait()
        k = g % NBUF
        buf = bufs[k]
        b, h = CHUNKS[g]
        prow0 = h * CHUNK

        @plsc.parallel_loop(0, CHUNK * COLV, 1, unroll=8)
        def _(i, buf=buf, prow0=prow0):
            sl = pl.ds(pl.multiple_of((i >> 5) << 4, LANES), LANES)
            r = i & (CHUNK - 1)
            plsc.addupdate(buf.at[r, sl], pos_v[prow0 + r, sl])

        if g + NBUF - 1 < NCHUNK:
            gh[g + NBUF - 1] = start_gather(g + NBUF - 1)
        out_base = b * S + p0 + h * CHUNK
        wb[k] = pltpu.async_copy(buf, out_hbm.at[pl.ds(out_base, CHUNK)],
                                 wsems[k])

    for k in range(NBUF):
        wb[k].wait()


def kernel(x, tok_table, pos_table):
    out = _emb_kernel(x.astype(jnp.int32), tok_table, pos_table)
    return out.reshape(B, S, D)


# speedup vs baseline: 1.0957x; 1.0304x over previous
import functools

import jax
import jax.numpy as jnp
from jax import lax
from jax.experimental import pallas as pl
from jax.experimental.pallas import tpu as pltpu
from jax.experimental.pallas import tpu_sc as plsc

VOCAB = 100000
D = 768
B = 4
S = 2048

NW = 32
P = S // NW
CHUNK = 16
NCHUNK = (B * P) // CHUNK
LANES = 16
COLV = D // LANES
RB = CHUNK.bit_length() - 1

_mesh = plsc.VectorSubcoreMesh(core_axis_name="c", subcore_axis_name="s")


NBUF = 4


@functools.partial(
    pl.kernel,
    mesh=_mesh,
    out_type=jax.ShapeDtypeStruct((B * S, D), jnp.float32),
    scratch_types=[
        pltpu.VMEM((B * P,), jnp.int32),
        pltpu.VMEM((P, D), jnp.float32),
        pltpu.VMEM((CHUNK, D), jnp.float32),
        pltpu.VMEM((CHUNK, D), jnp.float32),
        pltpu.VMEM((CHUNK, D), jnp.float32),
        pltpu.VMEM((CHUNK, D), jnp.float32),
        pltpu.SemaphoreType.DMA,
        pltpu.SemaphoreType.DMA,
        pltpu.SemaphoreType.DMA,
        pltpu.SemaphoreType.DMA,
        pltpu.SemaphoreType.DMA,
        pltpu.SemaphoreType.DMA,
        pltpu.SemaphoreType.DMA,
        pltpu.SemaphoreType.DMA,
        pltpu.SemaphoreType.DMA,
        pltpu.SemaphoreType.DMA,
        pltpu.SemaphoreType.DMA,
        pltpu.SemaphoreType.DMA,
        pltpu.SemaphoreType.DMA,
        pltpu.SemaphoreType.DMA,
    ],
)
def _emb_kernel(x_hbm, tok_hbm, pos_hbm, out_hbm,
                idx_v, pos_v, tok0, tok1, tok2, tok3,
                gsem0, gsem1, gsem2, gsem3, wsem0, wsem1, wsem2, wsem3,
                psem0, psem1,
                isem0, isem1, isem2, isem3):
    cid = lax.axis_index("c")
    sid = lax.axis_index("s")
    wid = sid * 2 + cid
    p0 = wid * P

    isems = (isem0, isem1, isem2, isem3)
    idx_hs = [pltpu.async_copy(x_hbm.at[b, pl.ds(p0, P)],
                               idx_v.at[pl.ds(b * P, P)], isems[b])
              for b in range(B)]
    HALF = P // 2
    pos_h0 = pltpu.async_copy(pos_hbm.at[pl.ds(p0, HALF)],
                              pos_v.at[pl.ds(0, HALF), :], psem0)
    pos_h1 = pltpu.async_copy(pos_hbm.at[pl.ds(p0 + HALF, HALF)],
                              pos_v.at[pl.ds(HALF, HALF), :], psem1)
    for h_ in idx_hs:
        h_.wait()

    bufs = (tok0, tok1, tok2, tok3)
    gsems = (gsem0, gsem1, gsem2, gsem3)
    wsems = (wsem0, wsem1, wsem2, wsem3)
    wb = [None] * NBUF
    gh = [None] * NCHUNK

    CHUNKS = [(b, h) for h in range(P // CHUNK) for b in range(B)]

    def start_gather(j):
        k = j % NBUF
        if wb[k] is not None:
            wb[k].wait()
        b, h = CHUNKS[j]
        return pltpu.async_copy(
            tok_hbm.at[idx_v.at[pl.ds((b * (P // CHUNK) + h) * CHUNK, CHUNK)]],
            bufs[k], gsems[k])

    for j in range(NBUF - 1):
        gh[j] = start_gather(j)
    pos_h0.wait()

    for g in range(NCHUNK):
        if g == NCHUNK // 2:
            pos_h1.wait()
        gh[g].wait()
        k = g % NBUF
        buf = bufs[k]
        b, h = CHUNKS[g]
        prow0 = h * CHUNK

        @plsc.parallel_loop(0, CHUNK * COLV, 1, unroll=8)
        def _(i, buf=buf, prow0=prow0):
            sl = pl.ds(pl.multiple_of((i >> RB) << 4, LANES), LANES)
            r = i & (CHUNK - 1)
            plsc.addupdate(buf.at[r, sl], pos_v[prow0 + r, sl])

        if g + NBUF - 1 < NCHUNK:
            gh[g + NBUF - 1] = start_gather(g + NBUF - 1)
        out_base = b * S + p0 + h * CHUNK
        wb[k] = pltpu.async_copy(buf, out_hbm.at[pl.ds(out_base, CHUNK)],
                                 wsems[k])

    for k in range(NBUF):
        wb[k].wait()


def kernel(x, tok_table, pos_table):
    out = _emb_kernel(x.astype(jnp.int32), tok_table, pos_table)
    return out.reshape(B, S, D)
